# baseline (speedup 1.0000x reference)
import jax
import jax.numpy as jnp
from jax.experimental import pallas as pl
from jax.experimental.pallas import tpu as pltpu

_C = 1024
_VC = 1000
_FMIN = -3.4e38


def _kl_kernel(k_ref, e_ref, out_ref, me_ref, ze_ref, s_ref, mk_ref, zk_ref):
    j = pl.program_id(1)

    @pl.when(j == 0)
    def _():
        me_ref[...] = jnp.full_like(me_ref, _FMIN)
        ze_ref[...] = jnp.zeros_like(ze_ref)
        s_ref[...] = jnp.zeros_like(s_ref)
        mk_ref[...] = jnp.full_like(mk_ref, _FMIN)
        zk_ref[...] = jnp.zeros_like(zk_ref)

    e = e_ref[...]
    k = k_ref[...]
    ones = jnp.ones((1, e.shape[0]), jnp.float32)

    def _colsum(x):
        return jax.lax.dot_general(
            ones, x, (((1,), (0,)), ((), ())),
            preferred_element_type=jnp.float32)

    m_e_old = me_ref[...]
    m_e = jnp.maximum(m_e_old, jnp.max(e, axis=0, keepdims=True))
    ee = jnp.exp(e - m_e)
    scale_e = jnp.exp(m_e_old - m_e)
    ze_ref[...] = ze_ref[...] * scale_e + _colsum(ee)
    s_ref[...] = s_ref[...] * scale_e + _colsum(ee * (e - k))
    me_ref[...] = m_e

    m_k_old = mk_ref[...]
    m_k = jnp.maximum(m_k_old, jnp.max(k, axis=0, keepdims=True))
    scale_k = jnp.exp(m_k_old - m_k)
    zk_ref[...] = zk_ref[...] * scale_k + _colsum(jnp.exp(k - m_k))
    mk_ref[...] = m_k

    @pl.when(j == pl.num_programs(1) - 1)
    def _():
        row = (s_ref[...] / ze_ref[...] - me_ref[...] - jnp.log(ze_ref[...])
               + mk_ref[...] + jnp.log(zk_ref[...]))
        out_ref[...] = jnp.full((1, 1, 128), jnp.sum(row), jnp.float32)


def kernel(korean_rep, english_rep):
    n_rows, vocab = korean_rep.shape
    kt = korean_rep.T
    et = english_rep.T
    n_col = n_rows // _C
    n_chunk = vocab // _VC
    out = pl.pallas_call(
        _kl_kernel,
        grid=(n_col, n_chunk),
        in_specs=[
            pl.BlockSpec((_VC, _C), lambda i, j: (j, i)),
            pl.BlockSpec((_VC, _C), lambda i, j: (j, i)),
        ],
        out_specs=pl.BlockSpec((1, 1, 128), lambda i, j: (i, 0, 0)),
        out_shape=jax.ShapeDtypeStruct((n_col, 1, 128), jnp.float32),
        scratch_shapes=[pltpu.VMEM((1, _C), jnp.float32) for _ in range(5)],
        compiler_params=pltpu.CompilerParams(
            dimension_semantics=("arbitrary", "arbitrary"),
            vmem_limit_bytes=60 * 1024 * 1024,
        ),
    )(kt, et)
    return jnp.sum(out[:, 0, 0]) / n_rows


# speedup vs baseline: 1.1133x; 1.1133x over previous
import jax
import jax.numpy as jnp
from jax.experimental import pallas as pl
from jax.experimental.pallas import tpu as pltpu

_C = 1024
_VC = 2000
_FMIN = -3.4e38


def _kl_kernel(k_ref, e_ref, out_ref, me_ref, ze_ref, s_ref, mk_ref, zk_ref):
    j = pl.program_id(0)

    @pl.when(j == 0)
    def _():
        me_ref[...] = jnp.full_like(me_ref, _FMIN)
        ze_ref[...] = jnp.zeros_like(ze_ref)
        s_ref[...] = jnp.zeros_like(s_ref)
        mk_ref[...] = jnp.full_like(mk_ref, _FMIN)
        zk_ref[...] = jnp.zeros_like(zk_ref)

    e = e_ref[...]
    k = k_ref[...]
    ones = jnp.ones((1, e.shape[0]), jnp.float32)

    def _colsum(x):
        return jax.lax.dot_general(
            ones, x, (((1,), (0,)), ((), ())),
            preferred_element_type=jnp.float32)

    m_e_old = me_ref[...]
    m_e = jnp.maximum(m_e_old, jnp.max(e, axis=0, keepdims=True))
    ee = jnp.exp(e - m_e)
    scale_e = jnp.exp(m_e_old - m_e)
    ze_ref[...] = ze_ref[...] * scale_e + _colsum(ee)
    s_ref[...] = s_ref[...] * scale_e + _colsum(ee * (e - k))
    me_ref[...] = m_e

    m_k_old = mk_ref[...]
    m_k = jnp.maximum(m_k_old, jnp.max(k, axis=0, keepdims=True))
    scale_k = jnp.exp(m_k_old - m_k)
    zk_ref[...] = zk_ref[...] * scale_k + _colsum(jnp.exp(k - m_k))
    mk_ref[...] = m_k

    @pl.when(j == pl.num_programs(0) - 1)
    def _():
        row = (s_ref[...] / ze_ref[...] - me_ref[...] - jnp.log(ze_ref[...])
               + mk_ref[...] + jnp.log(zk_ref[...]))
        out_ref[...] = jnp.full((1, 1, 128), jnp.sum(row), jnp.float32)


def kernel(korean_rep, english_rep):
    n_rows, vocab = korean_rep.shape
    kt = korean_rep.T
    et = english_rep.T
    n_chunk = vocab // _VC
    out = pl.pallas_call(
        _kl_kernel,
        grid=(n_chunk,),
        in_specs=[
            pl.BlockSpec((_VC, _C), lambda j: (j, 0)),
            pl.BlockSpec((_VC, _C), lambda j: (j, 0)),
        ],
        out_specs=pl.BlockSpec((1, 1, 128), lambda j: (0, 0, 0)),
        out_shape=jax.ShapeDtypeStruct((1, 1, 128), jnp.float32),
        scratch_shapes=[pltpu.VMEM((1, _C), jnp.float32) for _ in range(5)],
        compiler_params=pltpu.CompilerParams(
            dimension_semantics=("arbitrary",),
            vmem_limit_bytes=60 * 1024 * 1024,
        ),
    )(kt, et)
    return jnp.sum(out[:, 0, 0]) / n_rows


# speedup vs baseline: 1.1154x; 1.0019x over previous
import jax
import jax.numpy as jnp
from jax.experimental import pallas as pl
from jax.experimental.pallas import tpu as pltpu

_VC = 2000
_FMIN = -3.4e38


def _kl_kernel(k_ref, e_ref, out_ref, me_ref, ze_ref, s_ref, mk_ref, zk_ref):
    j = pl.program_id(0)

    @pl.when(j == 0)
    def _():
        me_ref[...] = jnp.full_like(me_ref, _FMIN)
        ze_ref[...] = jnp.zeros_like(ze_ref)
        s_ref[...] = jnp.zeros_like(s_ref)
        mk_ref[...] = jnp.full_like(mk_ref, _FMIN)
        zk_ref[...] = jnp.zeros_like(zk_ref)

    e = e_ref[...]
    k = k_ref[...]
    ones = jnp.ones((1, e.shape[0]), jnp.float32)

    def _colsum(x):
        return jax.lax.dot_general(
            ones, x, (((1,), (0,)), ((), ())),
            preferred_element_type=jnp.float32)

    m_e_old = me_ref[...]
    m_e = jnp.maximum(m_e_old, jnp.max(e, axis=0, keepdims=True))
    ee = jnp.exp(e - m_e)
    scale_e = jnp.exp(m_e_old - m_e)
    ze_ref[...] = ze_ref[...] * scale_e + _colsum(ee)
    s_ref[...] = s_ref[...] * scale_e + _colsum(ee * (e - k))
    me_ref[...] = m_e

    m_k_old = mk_ref[...]
    m_k = jnp.maximum(m_k_old, jnp.max(k, axis=0, keepdims=True))
    scale_k = jnp.exp(m_k_old - m_k)
    zk_ref[...] = zk_ref[...] * scale_k + _colsum(jnp.exp(k - m_k))
    mk_ref[...] = m_k

    @pl.when(j == pl.num_programs(0) - 1)
    def _():
        row = (s_ref[...] / ze_ref[...] - me_ref[...] - jnp.log(ze_ref[...])
               + mk_ref[...] + jnp.log(zk_ref[...]))
        out_ref[...] = jnp.full((1, 1, 128), jnp.sum(row), jnp.float32)


def kernel(korean_rep, english_rep):
    n_rows, vocab = korean_rep.shape
    kt = korean_rep.T
    et = english_rep.T
    n_chunk = vocab // _VC
    out = pl.pallas_call(
        _kl_kernel,
        grid=(n_chunk,),
        in_specs=[
            pl.BlockSpec((_VC, n_rows), lambda j: (j, 0)),
            pl.BlockSpec((_VC, n_rows), lambda j: (j, 0)),
        ],
        out_specs=pl.BlockSpec((1, 1, 128), lambda j: (0, 0, 0)),
        out_shape=jax.ShapeDtypeStruct((1, 1, 128), jnp.float32),
        scratch_shapes=[pltpu.VMEM((1, n_rows), jnp.float32) for _ in range(5)],
        compiler_params=pltpu.CompilerParams(
            dimension_semantics=("arbitrary",),
            vmem_limit_bytes=60 * 1024 * 1024,
        ),
    )(kt, et)
    return jnp.sum(out[:, 0, 0]) / n_rows
